# baseline (speedup 1.0000x reference)
import functools

import jax
import jax.numpy as jnp
from jax import lax
from jax.experimental import pallas as pl
from jax.experimental.pallas import tpu as pltpu
from jax.experimental.pallas import tpu_sc as plsc

B, N, C = 8, 2048, 32
K = 16
G = 32
BN = 512

NC, NS = 2, 16
NW = NC * NS
ROWS = B * N * K
PER_W = ROWS // NW
CH = 2048
NCH = PER_W // CH
SUB = CH // 128


def _knn_body(x_blk_ref, x_all_ref, w1b_ref, idx_ref, idxg_ref, p_ref):
    b = pl.program_id(0)
    nb = pl.program_id(1)
    x_blk = x_blk_ref[0]
    x_all = x_all_ref[0]
    dots2 = lax.dot_general(-2.0 * x_blk, x_all, (((1,), (1,)), ((), ())),
                            preferred_element_type=jnp.float32)
    a_sq = jnp.sum(x_blk * x_blk, axis=1, keepdims=True)
    b_sq = jnp.sum(x_all * x_all, axis=1)[None, :]
    d = (dots2 + a_sq) + b_sq
    colf = lax.broadcasted_iota(jnp.int32, (BN, N), 1).astype(jnp.float32)
    rowf = (lax.broadcasted_iota(jnp.int32, (BN, N), 0)
            + nb * BN).astype(jnp.float32)
    d = jnp.where(colf == rowf, jnp.inf, d)
    big = jnp.float32(2 * N)
    idx_cols = []
    for t in range(K):
        m = jnp.min(d, axis=1, keepdims=True)
        sel = jnp.min(jnp.where(d == m, colf, big), axis=1, keepdims=True)
        idx_cols.append(sel)
        if t < K - 1:
            d = jnp.where(colf == sel, jnp.inf, d)
    idx = jnp.concatenate(idx_cols, axis=1).astype(jnp.int32)
    idx_ref[0] = idx
    idxg_ref[0] = idx + b * N
    p_ref[0] = jnp.dot(x_blk, w1b_ref[...], preferred_element_type=jnp.float32)


def _knn_call(x, w1b, nb_):
    return pl.pallas_call(
        _knn_body,
        grid=(nb_, N // BN),
        in_specs=[
            pl.BlockSpec((1, BN, C), lambda b, n: (b, n, 0)),
            pl.BlockSpec((1, N, C), lambda b, n: (b, 0, 0)),
            pl.BlockSpec((C, G), lambda b, n: (0, 0)),
        ],
        out_specs=[
            pl.BlockSpec((1, BN, K), lambda b, n: (b, n, 0)),
            pl.BlockSpec((1, BN, K), lambda b, n: (b, n, 0)),
            pl.BlockSpec((1, BN, G), lambda b, n: (b, n, 0)),
        ],
        out_shape=[
            jax.ShapeDtypeStruct((nb_, N, K), jnp.int32),
            jax.ShapeDtypeStruct((nb_, N, K), jnp.int32),
            jax.ShapeDtypeStruct((nb_, N, G), jnp.float32),
        ],
    )(x, x, w1b)


def _make_gather_body(rows):
    per_w = rows // NW
    nch = max(per_w // CH, 1)
    ch_rows = per_w // nch

    def body(table_hbm, idx_hbm, out_hbm, idx_v, rows_v, sem):
        wid = lax.axis_index("s") * NC + lax.axis_index("c")
        idx_rows = per_w // 128
        pltpu.sync_copy(idx_hbm.at[pl.ds(wid * idx_rows, idx_rows)], idx_v)

        @pl.loop(0, nch)
        def _chunk(ch):
            copies = []
            for j in range(ch_rows // 128):
                copies.append(pltpu.async_copy(
                    table_hbm.at[idx_v.at[ch * (ch_rows // 128) + j]],
                    rows_v.at[pl.ds(j * 128, 128)], sem))
            for cp in copies:
                cp.wait()
            pltpu.sync_copy(
                rows_v, out_hbm.at[pl.ds(wid * per_w + ch * ch_rows, ch_rows)])

    return body, per_w, ch_rows


def _gather_call(table, idx2d, rows):
    mesh = plsc.VectorSubcoreMesh(core_axis_name="c", subcore_axis_name="s")
    body, per_w, ch_rows = _make_gather_body(rows)
    fn = pl.kernel(
        body,
        out_type=jax.ShapeDtypeStruct((rows, G), jnp.float32),
        mesh=mesh,
        compiler_params=pltpu.CompilerParams(use_tc_tiling_on_sc=False),
        scratch_types=[
            pltpu.VMEM((per_w // 128, 128), jnp.int32),
            pltpu.VMEM((ch_rows, G), jnp.float32),
            pltpu.SemaphoreType.DMA,
        ],
    )
    return fn(table, idx2d)


S = 4


def _conv_body(x_ref, g_ref, w1d_ref, b1_ref, w3a_ref, w3b_ref, b3_ref,
               w2a_ref, w2b_ref, w2c_ref, b2_ref, out_ref):
    x = x_ref[0]
    c1t = jnp.dot(x, w1d_ref[...], preferred_element_type=jnp.float32) + b1_ref[...]
    c3t = jnp.dot(x, w3b_ref[...], preferred_element_type=jnp.float32) + b3_ref[...]
    c2t = jnp.dot(x, w2c_ref[...], preferred_element_type=jnp.float32) + b2_ref[...]

    def smax(a):
        m = jnp.maximum(a[:, :2 * G], a[:, 2 * G:])
        return jnp.maximum(m[:, :G], m[:, G:])

    RT = 64
    for r in range(BN // RT):
        lo, hi = r * RT, (r + 1) * RT
        rs = pl.ds(lo, RT)
        c1r, c3r, c2r = c1t[lo:hi], c3t[lo:hi], c2t[lo:hi]
        neg = jnp.float32(-jnp.inf)
        acc1 = jnp.full((RT, S * G), neg, jnp.float32)
        acc3 = jnp.full((RT, S * G), neg, jnp.float32)
        acc2 = jnp.full((RT, S * G), neg, jnp.float32)
        for j in range(K // S):
            gj = g_ref[0, j, rs, :]
            y1 = jnp.maximum(gj + c1r, 0.0)
            y3 = jnp.maximum(
                jnp.dot(y1, w3a_ref[...], preferred_element_type=jnp.float32) + c3r,
                0.0)
            y2 = (jnp.dot(y3, w2a_ref[...], preferred_element_type=jnp.float32)
                  + jnp.dot(y1, w2b_ref[...], preferred_element_type=jnp.float32)
                  + c2r)
            acc1 = jnp.maximum(acc1, y1)
            acc3 = jnp.maximum(acc3, y3)
            acc2 = jnp.maximum(acc2, y2)
        out_ref[0, rs, :] = jnp.concatenate(
            [smax(acc2), smax(acc3), smax(acc1), x[lo:hi]], axis=1)


def _conv_call(x, g3, w1d, b1, w3a, w3b, b3, w2a, w2b, w2c, b2, nb_):
    tspec = pl.BlockSpec((C, S * G), lambda b, n: (0, 0))
    dspec = pl.BlockSpec((S * G, S * G), lambda b, n: (0, 0))
    bspec = pl.BlockSpec((1, S * G), lambda b, n: (0, 0))
    return pl.pallas_call(
        _conv_body,
        grid=(nb_, N // BN),
        in_specs=[
            pl.BlockSpec((1, BN, C), lambda b, n: (b, n, 0)),
            pl.BlockSpec((1, K // S, BN, S * G), lambda b, n: (b, 0, n, 0)),
            tspec, bspec, dspec, tspec, bspec, dspec, dspec, tspec, bspec,
        ],
        out_specs=pl.BlockSpec((1, BN, 4 * G), lambda b, n: (b, n, 0)),
        out_shape=jax.ShapeDtypeStruct((nb_, N, 4 * G), jnp.float32),
    )(x, g3, w1d, b1, w3a, w3b, b3, w2a, w2b, w2c, b2)


def kernel(inputs, W1, b1, W2, b2, W3, b3):
    x = inputs
    w1b = W1[C:]
    w1d = W1[:C] - w1b
    w3a, w3b = W3[:G], W3[G:]
    w2a, w2b, w2c = W2[:G], W2[G:2 * G], W2[2 * G:]

    eye = jnp.eye(S, dtype=jnp.float32)
    w3a_d = jnp.kron(eye, w3a)
    w2a_d = jnp.kron(eye, w2a)
    w2b_d = jnp.kron(eye, w2b)
    w1d_t = jnp.tile(w1d, (1, S))
    w3b_t = jnp.tile(w3b, (1, S))
    w2c_t = jnp.tile(w2c, (1, S))
    b1_t = jnp.tile(b1.reshape(1, G), (1, S))
    b3_t = jnp.tile(b3.reshape(1, G), (1, S))
    b2_t = jnp.tile(b2.reshape(1, G), (1, S))

    HB = B // 4
    ys, idxs = [], []
    for h in range(4):
        xh = lax.slice_in_dim(x, h * HB, (h + 1) * HB, axis=0)
        idx, idxg, p = _knn_call(xh, w1b, HB)
        rows = HB * N * K
        idxp = idxg.reshape(HB, N, K // S, S).transpose(0, 2, 1, 3)
        gathered = _gather_call(p.reshape(HB * N, G),
                                idxp.reshape(rows // 128, 128), rows)
        g3 = gathered.reshape(HB, K // S, N, S * G)
        yh = _conv_call(xh, g3, w1d_t, b1_t, w3a_d, w3b_t, b3_t,
                        w2a_d, w2b_d, w2c_t, b2_t, HB)
        ys.append(yh)
        idxs.append(idx)
    return jnp.concatenate(ys, axis=0), jnp.concatenate(idxs, axis=0)


# speedup vs baseline: 1.0364x; 1.0364x over previous
import functools

import jax
import jax.numpy as jnp
from jax import lax
from jax.experimental import pallas as pl
from jax.experimental.pallas import tpu as pltpu
from jax.experimental.pallas import tpu_sc as plsc

B, N, C = 8, 2048, 32
K = 16
G = 32
BN = 512

NC, NS = 2, 16
NW = NC * NS
ROWS = B * N * K
PER_W = ROWS // NW
CH = 2048
NCH = PER_W // CH
SUB = CH // 128


def _knn_body(x_blk_ref, x_all_ref, w1b_ref, idx_ref, idxg_ref, p_ref):
    b = pl.program_id(0)
    nb = pl.program_id(1)
    x_blk = x_blk_ref[0]
    x_all = x_all_ref[0]
    dots2 = lax.dot_general(-2.0 * x_blk, x_all, (((1,), (1,)), ((), ())),
                            preferred_element_type=jnp.float32)
    a_sq = jnp.sum(x_blk * x_blk, axis=1, keepdims=True)
    b_sq = jnp.sum(x_all * x_all, axis=1)[None, :]
    d = (dots2 + a_sq) + b_sq
    colf = lax.broadcasted_iota(jnp.int32, (BN, N), 1).astype(jnp.float32)
    rowf = (lax.broadcasted_iota(jnp.int32, (BN, N), 0)
            + nb * BN).astype(jnp.float32)
    d = jnp.where(colf == rowf, jnp.inf, d)
    big = jnp.float32(2 * N)
    idx_cols = []
    for t in range(K):
        m = jnp.min(d, axis=1, keepdims=True)
        sel = jnp.min(jnp.where(d == m, colf, big), axis=1, keepdims=True)
        idx_cols.append(sel)
        if t < K - 1:
            d = jnp.where(colf == sel, jnp.inf, d)
    idx = jnp.concatenate(idx_cols, axis=1).astype(jnp.int32)
    idx_ref[0] = idx
    idxg_ref[0] = idx + b * N
    p_ref[0] = jnp.dot(x_blk, w1b_ref[...], preferred_element_type=jnp.float32)


def _knn_call(x, w1b, nb_):
    return pl.pallas_call(
        _knn_body,
        grid=(nb_, N // BN),
        in_specs=[
            pl.BlockSpec((1, BN, C), lambda b, n: (b, n, 0)),
            pl.BlockSpec((1, N, C), lambda b, n: (b, 0, 0)),
            pl.BlockSpec((C, G), lambda b, n: (0, 0)),
        ],
        out_specs=[
            pl.BlockSpec((1, BN, K), lambda b, n: (b, n, 0)),
            pl.BlockSpec((1, BN, K), lambda b, n: (b, n, 0)),
            pl.BlockSpec((1, BN, G), lambda b, n: (b, n, 0)),
        ],
        out_shape=[
            jax.ShapeDtypeStruct((nb_, N, K), jnp.int32),
            jax.ShapeDtypeStruct((nb_, N, K), jnp.int32),
            jax.ShapeDtypeStruct((nb_, N, G), jnp.float32),
        ],
    )(x, x, w1b)


def _make_gather_body(rows):
    per_w = rows // NW
    nch = max(per_w // CH, 1)
    ch_rows = per_w // nch

    def body(table_hbm, idx_hbm, out_hbm, idx_v, rows_v, sem):
        wid = lax.axis_index("s") * NC + lax.axis_index("c")
        idx_rows = per_w // 128
        pltpu.sync_copy(idx_hbm.at[pl.ds(wid * idx_rows, idx_rows)], idx_v)

        @pl.loop(0, nch)
        def _chunk(ch):
            copies = []
            for j in range(ch_rows // 128):
                copies.append(pltpu.async_copy(
                    table_hbm.at[idx_v.at[ch * (ch_rows // 128) + j]],
                    rows_v.at[pl.ds(j * 128, 128)], sem))
            for cp in copies:
                cp.wait()
            pltpu.sync_copy(
                rows_v, out_hbm.at[pl.ds(wid * per_w + ch * ch_rows, ch_rows)])

    return body, per_w, ch_rows


def _gather_call(table, idx2d, rows):
    mesh = plsc.VectorSubcoreMesh(core_axis_name="c", subcore_axis_name="s")
    body, per_w, ch_rows = _make_gather_body(rows)
    fn = pl.kernel(
        body,
        out_type=jax.ShapeDtypeStruct((rows, G), jnp.float32),
        mesh=mesh,
        compiler_params=pltpu.CompilerParams(use_tc_tiling_on_sc=False),
        scratch_types=[
            pltpu.VMEM((per_w // 128, 128), jnp.int32),
            pltpu.VMEM((ch_rows, G), jnp.float32),
            pltpu.SemaphoreType.DMA,
        ],
    )
    return fn(table, idx2d)


S = 4


def _conv_body(x_ref, g_ref, w1d_ref, b1_ref, w3a_ref, w3b_ref, b3_ref,
               w2a_ref, w2b_ref, w2c_ref, b2_ref, out_ref):
    x = x_ref[0]
    c1t = jnp.dot(x, w1d_ref[...], preferred_element_type=jnp.float32) + b1_ref[...]
    c3t = jnp.dot(x, w3b_ref[...], preferred_element_type=jnp.float32) + b3_ref[...]
    c2t = jnp.dot(x, w2c_ref[...], preferred_element_type=jnp.float32) + b2_ref[...]

    def smax(a):
        m = jnp.maximum(a[:, :2 * G], a[:, 2 * G:])
        return jnp.maximum(m[:, :G], m[:, G:])

    RT = 64
    for r in range(BN // RT):
        lo, hi = r * RT, (r + 1) * RT
        rs = pl.ds(lo, RT)
        c1r, c3r, c2r = c1t[lo:hi], c3t[lo:hi], c2t[lo:hi]
        neg = jnp.float32(-jnp.inf)
        acc1 = jnp.full((RT, S * G), neg, jnp.float32)
        acc3 = jnp.full((RT, S * G), neg, jnp.float32)
        acc2 = jnp.full((RT, S * G), neg, jnp.float32)
        for j in range(K // S):
            gj = g_ref[0, j, rs, :]
            y1 = jnp.maximum(gj + c1r, 0.0)
            y3 = jnp.maximum(
                jnp.dot(y1, w3a_ref[...], preferred_element_type=jnp.float32) + c3r,
                0.0)
            y2 = (jnp.dot(y3, w2a_ref[...], preferred_element_type=jnp.float32)
                  + jnp.dot(y1, w2b_ref[...], preferred_element_type=jnp.float32)
                  + c2r)
            acc1 = jnp.maximum(acc1, y1)
            acc3 = jnp.maximum(acc3, y3)
            acc2 = jnp.maximum(acc2, y2)
        out_ref[0, rs, :] = jnp.concatenate(
            [smax(acc2), smax(acc3), smax(acc1), x[lo:hi]], axis=1)


def _conv_call(x, g3, w1d, b1, w3a, w3b, b3, w2a, w2b, w2c, b2, nb_):
    tspec = pl.BlockSpec((C, S * G), lambda b, n: (0, 0))
    dspec = pl.BlockSpec((S * G, S * G), lambda b, n: (0, 0))
    bspec = pl.BlockSpec((1, S * G), lambda b, n: (0, 0))
    return pl.pallas_call(
        _conv_body,
        grid=(nb_, N // BN),
        in_specs=[
            pl.BlockSpec((1, BN, C), lambda b, n: (b, n, 0)),
            pl.BlockSpec((1, K // S, BN, S * G), lambda b, n: (b, 0, n, 0)),
            tspec, bspec, dspec, tspec, bspec, dspec, dspec, tspec, bspec,
        ],
        out_specs=pl.BlockSpec((1, BN, 4 * G), lambda b, n: (b, n, 0)),
        out_shape=jax.ShapeDtypeStruct((nb_, N, 4 * G), jnp.float32),
    )(x, g3, w1d, b1, w3a, w3b, b3, w2a, w2b, w2c, b2)


def kernel(inputs, W1, b1, W2, b2, W3, b3):
    x = inputs
    w1b = W1[C:]
    w1d = W1[:C] - w1b
    w3a, w3b = W3[:G], W3[G:]
    w2a, w2b, w2c = W2[:G], W2[G:2 * G], W2[2 * G:]

    eye = jnp.eye(S, dtype=jnp.float32)
    w3a_d = jnp.kron(eye, w3a)
    w2a_d = jnp.kron(eye, w2a)
    w2b_d = jnp.kron(eye, w2b)
    w1d_t = jnp.tile(w1d, (1, S))
    w3b_t = jnp.tile(w3b, (1, S))
    w2c_t = jnp.tile(w2c, (1, S))
    b1_t = jnp.tile(b1.reshape(1, G), (1, S))
    b3_t = jnp.tile(b3.reshape(1, G), (1, S))
    b2_t = jnp.tile(b2.reshape(1, G), (1, S))

    HB = B // 2
    ys, idxs = [], []
    for h in range(2):
        xh = lax.slice_in_dim(x, h * HB, (h + 1) * HB, axis=0)
        idx, idxg, p = _knn_call(xh, w1b, HB)
        rows = HB * N * K
        idxp = idxg.reshape(HB, N, K // S, S).transpose(0, 2, 1, 3)
        gathered = _gather_call(p.reshape(HB * N, G),
                                idxp.reshape(rows // 128, 128), rows)
        g3 = gathered.reshape(HB, K // S, N, S * G)
        yh = _conv_call(xh, g3, w1d_t, b1_t, w3a_d, w3b_t, b3_t,
                        w2a_d, w2b_d, w2c_t, b2_t, HB)
        ys.append(yh)
        idxs.append(idx)
    return jnp.concatenate(ys, axis=0), jnp.concatenate(idxs, axis=0)


# speedup vs baseline: 1.0367x; 1.0003x over previous
import functools

import jax
import jax.numpy as jnp
from jax import lax
from jax.experimental import pallas as pl
from jax.experimental.pallas import tpu as pltpu
from jax.experimental.pallas import tpu_sc as plsc

B, N, C = 8, 2048, 32
K = 16
G = 32
BN = 512

NC, NS = 2, 16
NW = NC * NS
ROWS = B * N * K
PER_W = ROWS // NW
CH = 2048
NCH = PER_W // CH
SUB = CH // 128


def _knn_body(x_blk_ref, x_all_ref, w1b_ref, idx_ref, idxg_ref, p_ref):
    b = pl.program_id(0)
    nb = pl.program_id(1)
    x_blk = x_blk_ref[0]
    x_all = x_all_ref[0]
    dots2 = lax.dot_general(-2.0 * x_blk, x_all, (((1,), (1,)), ((), ())),
                            preferred_element_type=jnp.float32)
    a_sq = jnp.sum(x_blk * x_blk, axis=1, keepdims=True)
    b_sq = jnp.sum(x_all * x_all, axis=1)[None, :]
    d = (dots2 + a_sq) + b_sq
    colf = lax.broadcasted_iota(jnp.int32, (BN, N), 1).astype(jnp.float32)
    rowf = (lax.broadcasted_iota(jnp.int32, (BN, N), 0)
            + nb * BN).astype(jnp.float32)
    d = jnp.where(colf == rowf, jnp.inf, d)
    big = jnp.float32(2 * N)
    idx_cols = []
    for t in range(K):
        m = jnp.min(d, axis=1, keepdims=True)
        sel = jnp.min(jnp.where(d == m, colf, big), axis=1, keepdims=True)
        idx_cols.append(sel)
        if t < K - 1:
            d = jnp.where(colf == sel, jnp.inf, d)
    idx = jnp.concatenate(idx_cols, axis=1).astype(jnp.int32)
    idx_ref[0] = idx
    idxg_ref[0] = idx + b * N
    p_ref[0] = jnp.dot(x_blk, w1b_ref[...], preferred_element_type=jnp.float32)


def _knn_call(x, w1b, nb_):
    return pl.pallas_call(
        _knn_body,
        grid=(nb_, N // BN),
        in_specs=[
            pl.BlockSpec((1, BN, C), lambda b, n: (b, n, 0)),
            pl.BlockSpec((1, N, C), lambda b, n: (b, 0, 0)),
            pl.BlockSpec((C, G), lambda b, n: (0, 0)),
        ],
        out_specs=[
            pl.BlockSpec((1, BN, K), lambda b, n: (b, n, 0)),
            pl.BlockSpec((1, BN, K), lambda b, n: (b, n, 0)),
            pl.BlockSpec((1, BN, G), lambda b, n: (b, n, 0)),
        ],
        out_shape=[
            jax.ShapeDtypeStruct((nb_, N, K), jnp.int32),
            jax.ShapeDtypeStruct((nb_, N, K), jnp.int32),
            jax.ShapeDtypeStruct((nb_, N, G), jnp.float32),
        ],
    )(x, x, w1b)


def _make_gather_body(rows):
    per_w = rows // NW
    nch = max(per_w // CH, 1)
    ch_rows = per_w // nch

    def body(table_hbm, idx_hbm, out_hbm, idx_v, rows_v, sem):
        wid = lax.axis_index("s") * NC + lax.axis_index("c")
        idx_rows = per_w // 128
        pltpu.sync_copy(idx_hbm.at[pl.ds(wid * idx_rows, idx_rows)], idx_v)

        @pl.loop(0, nch)
        def _chunk(ch):
            copies = []
            for j in range(ch_rows // 128):
                copies.append(pltpu.async_copy(
                    table_hbm.at[idx_v.at[ch * (ch_rows // 128) + j]],
                    rows_v.at[pl.ds(j * 128, 128)], sem))
            for cp in copies:
                cp.wait()
            pltpu.sync_copy(
                rows_v, out_hbm.at[pl.ds(wid * per_w + ch * ch_rows, ch_rows)])

    return body, per_w, ch_rows


def _gather_call(table, idx2d, rows):
    mesh = plsc.VectorSubcoreMesh(core_axis_name="c", subcore_axis_name="s")
    body, per_w, ch_rows = _make_gather_body(rows)
    fn = pl.kernel(
        body,
        out_type=jax.ShapeDtypeStruct((rows, G), jnp.float32),
        mesh=mesh,
        compiler_params=pltpu.CompilerParams(use_tc_tiling_on_sc=False),
        scratch_types=[
            pltpu.VMEM((per_w // 128, 128), jnp.int32),
            pltpu.VMEM((ch_rows, G), jnp.float32),
            pltpu.SemaphoreType.DMA,
        ],
    )
    return fn(table, idx2d)


S = 4


def _conv_body(x_ref, g_ref, w1d_ref, b1_ref, w3a_ref, w3b_ref, b3_ref,
               w2a_ref, w2b_ref, w2c_ref, b2_ref, out_ref):
    x = x_ref[0]
    c1t = jnp.dot(x, w1d_ref[...], preferred_element_type=jnp.float32) + b1_ref[...]
    c3t = jnp.dot(x, w3b_ref[...], preferred_element_type=jnp.float32) + b3_ref[...]
    c2t = jnp.dot(x, w2c_ref[...], preferred_element_type=jnp.float32) + b2_ref[...]

    def smax(a):
        m = jnp.maximum(a[:, :2 * G], a[:, 2 * G:])
        return jnp.maximum(m[:, :G], m[:, G:])

    RT = 64
    for r in range(BN // RT):
        lo, hi = r * RT, (r + 1) * RT
        rs = pl.ds(lo, RT)
        c1r, c3r, c2r = c1t[lo:hi], c3t[lo:hi], c2t[lo:hi]
        neg = jnp.float32(-jnp.inf)
        acc1 = jnp.full((RT, S * G), neg, jnp.float32)
        acc3 = jnp.full((RT, S * G), neg, jnp.float32)
        acc2 = jnp.full((RT, S * G), neg, jnp.float32)
        for j in range(K // S):
            gj = g_ref[0, j, rs, :]
            y1 = jnp.maximum(gj + c1r, 0.0)
            y3 = jnp.maximum(
                jnp.dot(y1, w3a_ref[...], preferred_element_type=jnp.float32) + c3r,
                0.0)
            y2 = (jnp.dot(y3, w2a_ref[...], preferred_element_type=jnp.float32)
                  + jnp.dot(y1, w2b_ref[...], preferred_element_type=jnp.float32)
                  + c2r)
            acc1 = jnp.maximum(acc1, y1)
            acc3 = jnp.maximum(acc3, y3)
            acc2 = jnp.maximum(acc2, y2)
        out_ref[0, rs, :] = jnp.concatenate(
            [smax(acc2), smax(acc3), smax(acc1), x[lo:hi]], axis=1)


def _conv_call(x, g3, w1d, b1, w3a, w3b, b3, w2a, w2b, w2c, b2, nb_):
    tspec = pl.BlockSpec((C, S * G), lambda b, n: (0, 0))
    dspec = pl.BlockSpec((S * G, S * G), lambda b, n: (0, 0))
    bspec = pl.BlockSpec((1, S * G), lambda b, n: (0, 0))
    return pl.pallas_call(
        _conv_body,
        grid=(nb_, N // BN),
        in_specs=[
            pl.BlockSpec((1, BN, C), lambda b, n: (b, n, 0)),
            pl.BlockSpec((1, K // S, BN, S * G), lambda b, n: (b, 0, n, 0)),
            tspec, bspec, dspec, tspec, bspec, dspec, dspec, tspec, bspec,
        ],
        out_specs=pl.BlockSpec((1, BN, 4 * G), lambda b, n: (b, n, 0)),
        out_shape=jax.ShapeDtypeStruct((nb_, N, 4 * G), jnp.float32),
    )(x, g3, w1d, b1, w3a, w3b, b3, w2a, w2b, w2c, b2)


def kernel(inputs, W1, b1, W2, b2, W3, b3):
    x = inputs
    w1b = W1[C:]
    w1d = W1[:C] - w1b
    w3a, w3b = W3[:G], W3[G:]
    w2a, w2b, w2c = W2[:G], W2[G:2 * G], W2[2 * G:]

    eye = jnp.eye(S, dtype=jnp.float32)
    w3a_d = jnp.kron(eye, w3a)
    w2a_d = jnp.kron(eye, w2a)
    w2b_d = jnp.kron(eye, w2b)
    w1d_t = jnp.tile(w1d, (1, S))
    w3b_t = jnp.tile(w3b, (1, S))
    w2c_t = jnp.tile(w2c, (1, S))
    b1_t = jnp.tile(b1.reshape(1, G), (1, S))
    b3_t = jnp.tile(b3.reshape(1, G), (1, S))
    b2_t = jnp.tile(b2.reshape(1, G), (1, S))

    HB = B // 2
    rows = HB * N * K
    xs, idxs, gs = [], [], []
    for h in range(2):
        xh = lax.slice_in_dim(x, h * HB, (h + 1) * HB, axis=0)
        xs.append(xh)
        idx, idxg, p = _knn_call(xh, w1b, HB)
        idxs.append(idx)
        idxp = idxg.reshape(HB, N, K // S, S).transpose(0, 2, 1, 3)
        gathered = _gather_call(p.reshape(HB * N, G),
                                idxp.reshape(rows // 128, 128), rows)
        gs.append(gathered.reshape(HB, K // S, N, S * G))
    ys = [_conv_call(xs[h], gs[h], w1d_t, b1_t, w3a_d, w3b_t, b3_t,
                     w2a_d, w2b_d, w2c_t, b2_t, HB) for h in range(2)]
    return jnp.concatenate(ys, axis=0), jnp.concatenate(idxs, axis=0)
